# baseline (speedup 1.0000x reference)
import functools

import jax
import jax.numpy as jnp
import numpy as np
from jax import lax
from jax.experimental import pallas as pl
from jax.experimental.pallas import tpu as pltpu
from jax.experimental.pallas import tpu_sc as plsc

N = 10000
E = 320000
EPS = 1e-7

NC = 2
NS = 16
NW = NC * NS
CHUNK = 80
NCHUNK = 125
EPAD = NW * NCHUNK * CHUNK - E
NP = 10080
PAD_DST = 10040
ROWS_PER_SUB = NP // NS
ZROWS = 126

assert NW * CHUNK * NCHUNK == E + EPAD
assert EPAD == 0 or N <= PAD_DST < NP
assert ZROWS * 5 == ROWS_PER_SUB
assert (NCHUNK - 5) % 4 == 0 and NCHUNK >= 9


@functools.lru_cache(maxsize=None)
def _make_sc_scatter(F, nsplit, with_deg):
    mesh = plsc.VectorSubcoreMesh(core_axis_name="c", subcore_axis_name="s",
                                  num_cores=NC, num_subcores=NS)
    out_type = [jax.ShapeDtypeStruct((NC, NP, F), jnp.float32)
                for _ in range(nsplit)]
    if with_deg:
        out_type.append(jax.ShapeDtypeStruct((NW, NP), jnp.float32))

    scratch_seq = [
        pltpu.VMEM((NCHUNK, CHUNK), jnp.int32),
        pltpu.VMEM((NCHUNK, CHUNK), jnp.int32),
        pltpu.VMEM((CHUNK, F), jnp.float32),
        pltpu.VMEM((CHUNK, F), jnp.float32),
        pltpu.VMEM((CHUNK, F), jnp.float32),
        pltpu.VMEM((CHUNK, F), jnp.float32),
        pltpu.VMEM((ZROWS, F), jnp.float32),
        pltpu.VMEM_SHARED((NP, F), jnp.float32),
        pltpu.SemaphoreType.DMA,
        pltpu.SemaphoreType.DMA,
        pltpu.SemaphoreType.DMA,
        pltpu.SemaphoreType.DMA,
        pltpu.SemaphoreType.DMA,
        pltpu.SemaphoreType.DMA,
        pltpu.SemaphoreType.DMA,
        pltpu.SemaphoreType.DMA,
    ]
    if with_deg:
        scratch_seq.append(pltpu.VMEM((NP,), jnp.float32))

    def tec_body(*refs):
        m_hbm = refs[:nsplit]
        src_hbm, dst_hbm = refs[nsplit], refs[nsplit + 1]
        s_out = refs[nsplit + 2:2 * nsplit + 2]
        rest = refs[2 * nsplit + 2:]
        if with_deg:
            deg_out = rest[0]
            rest = rest[1:]
        (src_v, dst_v, b0, b1, b2, b3, zbuf, s_sh,
         g0, g1, g2, g3, s0, s1, s2, s3) = rest[:16]
        bufs = [b0, b1, b2, b3]
        gsems = [g0, g1, g2, g3]
        ssems = [s0, s1, s2, s3]
        if with_deg:
            hist = rest[16]

        cid = lax.axis_index("c")
        sid = lax.axis_index("s")
        wid = cid * NS + sid
        base = sid * ROWS_PER_SUB

        zero16 = jnp.zeros((16,), jnp.float32)
        one16 = jnp.ones((16,), jnp.float32)

        def zb_row(r, _):
            for cc in range(F // 16):
                zbuf[r, pl.ds(cc * 16, 16)] = zero16
            return 0
        lax.fori_loop(0, ZROWS, zb_row, 0)
        if with_deg:
            def h_row(r, _):
                hist[pl.ds(r * 16, 16)] = zero16
                return 0
            lax.fori_loop(0, NP // 16, h_row, 0)

        pltpu.sync_copy(src_hbm.at[wid], src_v)
        pltpu.sync_copy(dst_hbm.at[wid], dst_v)

        for p in range(nsplit):
            first = p == 0
            for t in range(5):
                pltpu.sync_copy(zbuf, s_sh.at[pl.ds(base + t * ZROWS, ZROWS)])
            plsc.subcore_barrier()

            m = m_hbm[p]

            def g_start(c, b):
                pltpu.async_copy(m.at[src_v.at[c]], bufs[b], gsems[b])

            def g_wait(c, b):
                pltpu.make_async_copy(m.at[src_v.at[c]], bufs[b],
                                      gsems[b]).wait()

            def s_start(c, b):
                pltpu.async_copy(bufs[b], s_sh.at[dst_v.at[c]], ssems[b],
                                 add=True)

            def s_wait(c, b):
                pltpu.make_async_copy(bufs[b], s_sh.at[dst_v.at[c]],
                                      ssems[b]).wait()

            def consume(j, b):
                g_wait(j, b)
                s_start(j, b)
                if with_deg and first:
                    for k in range(CHUNK // 16):
                        iv = dst_v[j, pl.ds(k * 16, 16)]
                        plsc.addupdate_scatter(hist, [iv], one16)

            g_start(0, 0)
            g_start(1, 1)
            g_start(2, 2)
            consume(0, 0)
            g_start(3, 3)
            consume(1, 1)

            def quad(k, _):
                j0 = 4 * k + 2
                for off in range(4):
                    j = j0 + off
                    b = (2 + off) % 4
                    bn = (b + 2) % 4
                    s_wait(j - 2, bn)
                    g_start(j + 2, bn)
                    consume(j, b)
                return 0
            lax.fori_loop(0, (NCHUNK - 5) // 4, quad, 0)

            for j in (NCHUNK - 3, NCHUNK - 2, NCHUNK - 1):
                b = j % 4
                bn = (b + 2) % 4
                s_wait(j - 2, bn)
                if j + 2 < NCHUNK:
                    g_start(j + 2, bn)
                consume(j, b)
            s_wait(NCHUNK - 2, (NCHUNK - 2) % 4)
            s_wait(NCHUNK - 1, (NCHUNK - 1) % 4)
            plsc.subcore_barrier()

            pltpu.sync_copy(s_sh.at[pl.ds(base, ROWS_PER_SUB)],
                            s_out[p].at[cid, pl.ds(base, ROWS_PER_SUB)])
        if with_deg:
            pltpu.sync_copy(hist, deg_out.at[wid])

    return pl.kernel(
        tec_body,
        out_type=tuple(out_type),
        mesh=mesh,
        scratch_types=tuple(scratch_seq),
        compiler_params=pltpu.CompilerParams(
            use_tc_tiling_on_sc=False,
            needs_layout_passes=not with_deg),
    )


_BLK = 2000
_GRID = N // _BLK
_BN_INV = 1.0 / np.sqrt(1.0 + 1e-5)


def _msg_body(x_ref, lo_ref, hi_ref):
    m = jnp.clip(jnp.maximum(x_ref[...], 0.0) + EPS, 1e-7, 1e1)
    lo_ref[...] = m[:, :64]
    hi_ref[...] = m[:, 64:]


_msg_tc = pl.pallas_call(
    _msg_body,
    grid=(_GRID,),
    in_specs=[pl.BlockSpec((_BLK, 128), lambda i: (i, 0))],
    out_specs=[pl.BlockSpec((_BLK, 64), lambda i: (i, 0)),
               pl.BlockSpec((_BLK, 64), lambda i: (i, 0))],
    out_shape=[jax.ShapeDtypeStruct((N, 64), jnp.float32),
               jax.ShapeDtypeStruct((N, 64), jnp.float32)],
)


def _make_layer_tc(F, Fh, Fo, last, nsplit=1):
    def body(*refs):
        sp = refs[:nsplit]
        (degp_ref, x_ref, wa_ref, ba_ref, g_ref, be_ref, wb_ref,
         bb_ref) = refs[nsplit:nsplit + 8]
        out_refs = refs[nsplit + 8:]
        s = jnp.concatenate([r[0] + r[1] for r in sp], axis=-1)
        deg = jnp.sum(degp_ref[...], axis=1, keepdims=True)
        deg = jnp.maximum(deg, 1.0)
        out = jnp.clip(s / deg, 1e-7, 1e1) + x_ref[...]
        h = jnp.dot(out, wa_ref[...], preferred_element_type=jnp.float32)
        h = (h + ba_ref[...]) * (g_ref[...] * _BN_INV) + be_ref[...]
        h = jnp.maximum(h, 0.0)
        o = jnp.dot(h, wb_ref[...], preferred_element_type=jnp.float32)
        o = jnp.maximum(o + bb_ref[...], 0.0)
        out_refs[0][...] = o
        if not last:
            out_refs[1][...] = jnp.clip(o + EPS, 1e-7, 1e1)

    Fs = F // nsplit
    out_shape = [jax.ShapeDtypeStruct((N, Fo), jnp.float32)]
    out_specs = [pl.BlockSpec((_BLK, Fo), lambda i: (i, 0))]
    if not last:
        out_shape.append(jax.ShapeDtypeStruct((N, Fo), jnp.float32))
        out_specs.append(pl.BlockSpec((_BLK, Fo), lambda i: (i, 0)))

    return pl.pallas_call(
        body,
        grid=(_GRID,),
        in_specs=(
            [pl.BlockSpec((2, _BLK, Fs), lambda i: (0, i, 0))] * nsplit +
            [
                pl.BlockSpec((_BLK, NW), lambda i: (i, 0)),
                pl.BlockSpec((_BLK, F), lambda i: (i, 0)),
                pl.BlockSpec((F, Fh), lambda i: (0, 0)),
                pl.BlockSpec((1, Fh), lambda i: (0, 0)),
                pl.BlockSpec((1, Fh), lambda i: (0, 0)),
                pl.BlockSpec((1, Fh), lambda i: (0, 0)),
                pl.BlockSpec((Fh, Fo), lambda i: (0, 0)),
                pl.BlockSpec((1, Fo), lambda i: (0, 0)),
            ]
        ),
        out_specs=out_specs,
        out_shape=out_shape,
    )


_layer1_tc = _make_layer_tc(128, 256, 32, last=False, nsplit=2)
_layer2_tc = _make_layer_tc(32, 64, 16, last=False)
_layer3_tc = _make_layer_tc(16, 32, 8, last=True)


def kernel(x, edge_index, w1a, b1a, g1, be1, w1b, b1b,
           w2a, b2a, g2, be2, w2b, b2b, w3a, b3a, g3, be3, w3b, b3b):
    if EPAD:
        src = jnp.concatenate(
            [edge_index[0], jnp.zeros((EPAD,), jnp.int32)])
        dst = jnp.concatenate(
            [edge_index[1], jnp.full((EPAD,), PAD_DST, jnp.int32)])
    else:
        src, dst = edge_index[0], edge_index[1]
    src = src.reshape(NW, NCHUNK, CHUNK)
    dst = dst.reshape(NW, NCHUNK, CHUNK)

    r2 = lambda v: v.reshape(1, -1)

    m1lo, m1hi = _msg_tc(x)
    s1lo, s1hi, degh = _make_sc_scatter(64, 2, True)(m1lo, m1hi, src, dst)
    degp = jnp.swapaxes(degh, 0, 1)
    h1, m2 = _layer1_tc(s1lo, s1hi, degp, x, w1a, r2(b1a), r2(g1), r2(be1),
                        w1b, r2(b1b))
    s2p = _make_sc_scatter(32, 1, False)(m2, src, dst)[0]
    h2, m3 = _layer2_tc(s2p, degp, h1, w2a, r2(b2a), r2(g2), r2(be2),
                        w2b, r2(b2b))
    s3p = _make_sc_scatter(16, 1, False)(m3, src, dst)[0]
    h3 = _layer3_tc(s3p, degp, h2, w3a, r2(b3a), r2(g3), r2(be3),
                    w3b, r2(b3b))[0]
    return h3


# speedup vs baseline: 1.1166x; 1.1166x over previous
import functools

import jax
import jax.numpy as jnp
import numpy as np
from jax import lax
from jax.experimental import pallas as pl
from jax.experimental.pallas import tpu as pltpu
from jax.experimental.pallas import tpu_sc as plsc

N = 10000
E = 320000
EPS = 1e-7

NC = 2
NS = 16
NW = NC * NS
CHUNK = 80
NCHUNK = 125
EPAD = NW * NCHUNK * CHUNK - E
NP = 10240
PAD_DST = 10200
ROWS_PER_SUB = NP // NS
ZROWS = 128

assert NW * CHUNK * NCHUNK == E + EPAD
assert EPAD == 0 or N <= PAD_DST < NP
assert ZROWS * 5 == ROWS_PER_SUB
assert (NCHUNK - 5) % 4 == 0 and NCHUNK >= 9


@functools.lru_cache(maxsize=None)
def _make_sc_scatter(F, nsplit, with_deg):
    mesh = plsc.VectorSubcoreMesh(core_axis_name="c", subcore_axis_name="s",
                                  num_cores=NC, num_subcores=NS)
    out_type = [jax.ShapeDtypeStruct((NC, NP, F * nsplit), jnp.float32)]
    if with_deg:
        out_type.append(jax.ShapeDtypeStruct((NW, NP), jnp.float32))

    scratch_seq = [
        pltpu.VMEM((NCHUNK, CHUNK), jnp.int32),
        pltpu.VMEM((NCHUNK, CHUNK), jnp.int32),
        pltpu.VMEM((CHUNK, F), jnp.float32),
        pltpu.VMEM((CHUNK, F), jnp.float32),
        pltpu.VMEM((CHUNK, F), jnp.float32),
        pltpu.VMEM((CHUNK, F), jnp.float32),
        pltpu.VMEM((ZROWS, F), jnp.float32),
        pltpu.VMEM_SHARED((NP, F), jnp.float32),
        pltpu.SemaphoreType.DMA,
        pltpu.SemaphoreType.DMA,
        pltpu.SemaphoreType.DMA,
        pltpu.SemaphoreType.DMA,
        pltpu.SemaphoreType.DMA,
        pltpu.SemaphoreType.DMA,
        pltpu.SemaphoreType.DMA,
        pltpu.SemaphoreType.DMA,
    ]
    if with_deg:
        scratch_seq.append(pltpu.VMEM((NP,), jnp.float32))

    def tec_body(*refs):
        m_hbm = refs[:nsplit]
        eidx_hbm = refs[nsplit]
        s_out = refs[nsplit + 1]
        rest = refs[nsplit + 2:]
        if with_deg:
            deg_out = rest[0]
            rest = rest[1:]
        (src_v, dst_v, b0, b1, b2, b3, zbuf, s_sh,
         g0, g1, g2, g3, s0, s1, s2, s3) = rest[:16]
        bufs = [b0, b1, b2, b3]
        gsems = [g0, g1, g2, g3]
        ssems = [s0, s1, s2, s3]
        if with_deg:
            hist = rest[16]

        cid = lax.axis_index("c")
        sid = lax.axis_index("s")
        wid = cid * NS + sid
        base = sid * ROWS_PER_SUB

        zero16 = jnp.zeros((16,), jnp.float32)
        one16 = jnp.ones((16,), jnp.float32)

        def zb_row(r, _):
            for cc in range(F // 16):
                zbuf[r, pl.ds(cc * 16, 16)] = zero16
            return 0
        lax.fori_loop(0, ZROWS, zb_row, 0)
        if with_deg:
            def h_row(r, _):
                hist[pl.ds(r * 16, 16)] = zero16
                return 0
            lax.fori_loop(0, NP // 16, h_row, 0)

        pltpu.sync_copy(eidx_hbm.at[0, wid], src_v)
        pltpu.sync_copy(eidx_hbm.at[1, wid], dst_v)

        for p in range(nsplit):
            first = p == 0
            for t in range(5):
                pltpu.sync_copy(zbuf, s_sh.at[pl.ds(base + t * ZROWS, ZROWS)])
            plsc.subcore_barrier()

            m = m_hbm[p]

            def g_start(c, b):
                pltpu.async_copy(m.at[src_v.at[c]], bufs[b], gsems[b])

            def g_wait(c, b):
                pltpu.make_async_copy(m.at[src_v.at[c]], bufs[b],
                                      gsems[b]).wait()

            def s_start(c, b):
                pltpu.async_copy(bufs[b], s_sh.at[dst_v.at[c]], ssems[b],
                                 add=True)

            def s_wait(c, b):
                pltpu.make_async_copy(bufs[b], s_sh.at[dst_v.at[c]],
                                      ssems[b]).wait()

            def consume(j, b):
                g_wait(j, b)
                s_start(j, b)
                if with_deg and first:
                    for k in range(CHUNK // 16):
                        iv = dst_v[j, pl.ds(k * 16, 16)]
                        plsc.addupdate_scatter(hist, [iv], one16)

            g_start(0, 0)
            g_start(1, 1)
            g_start(2, 2)
            consume(0, 0)
            g_start(3, 3)
            consume(1, 1)

            def quad(k, _):
                j0 = 4 * k + 2
                for off in range(4):
                    j = j0 + off
                    b = (2 + off) % 4
                    bn = (b + 2) % 4
                    s_wait(j - 2, bn)
                    g_start(j + 2, bn)
                    consume(j, b)
                return 0
            lax.fori_loop(0, (NCHUNK - 5) // 4, quad, 0)

            for j in (NCHUNK - 3, NCHUNK - 2, NCHUNK - 1):
                b = j % 4
                bn = (b + 2) % 4
                s_wait(j - 2, bn)
                if j + 2 < NCHUNK:
                    g_start(j + 2, bn)
                consume(j, b)
            s_wait(NCHUNK - 2, (NCHUNK - 2) % 4)
            s_wait(NCHUNK - 1, (NCHUNK - 1) % 4)
            plsc.subcore_barrier()

            if nsplit == 1:
                dst_ref = s_out.at[cid, pl.ds(base, ROWS_PER_SUB)]
            else:
                dst_ref = s_out.at[cid, pl.ds(base, ROWS_PER_SUB),
                                   pl.ds(p * F, F)]
            pltpu.sync_copy(s_sh.at[pl.ds(base, ROWS_PER_SUB)], dst_ref)
        if with_deg:
            pltpu.sync_copy(hist, deg_out.at[wid])

    return pl.kernel(
        tec_body,
        out_type=tuple(out_type),
        mesh=mesh,
        scratch_types=tuple(scratch_seq),
        compiler_params=pltpu.CompilerParams(
            use_tc_tiling_on_sc=False,
            needs_layout_passes=not with_deg),
    )


_BLK = 2000
_GRID = N // _BLK
_BN_INV = 1.0 / np.sqrt(1.0 + 1e-5)


def _msg_body(x_ref, lo_ref, hi_ref):
    m = jnp.clip(jnp.maximum(x_ref[...], 0.0) + EPS, 1e-7, 1e1)
    lo_ref[...] = m[:, :64]
    hi_ref[...] = m[:, 64:]


_msg_tc = pl.pallas_call(
    _msg_body,
    grid=(_GRID,),
    in_specs=[pl.BlockSpec((_BLK, 128), lambda i: (i, 0))],
    out_specs=[pl.BlockSpec((_BLK, 64), lambda i: (i, 0)),
               pl.BlockSpec((_BLK, 64), lambda i: (i, 0))],
    out_shape=[jax.ShapeDtypeStruct((N, 64), jnp.float32),
               jax.ShapeDtypeStruct((N, 64), jnp.float32)],
)


def _make_layer_tc(F, Fh, Fo, last, nsplit=1):
    def body(*refs):
        sp = refs[:nsplit]
        (degp_ref, x_ref, wa_ref, ba_ref, g_ref, be_ref, wb_ref,
         bb_ref) = refs[nsplit:nsplit + 8]
        out_refs = refs[nsplit + 8:]
        s = jnp.concatenate([r[0] + r[1] for r in sp], axis=-1)
        deg = jnp.sum(degp_ref[...], axis=1, keepdims=True)
        deg = jnp.maximum(deg, 1.0)
        out = jnp.clip(s / deg, 1e-7, 1e1) + x_ref[...]
        h = jnp.dot(out, wa_ref[...], preferred_element_type=jnp.float32)
        h = (h + ba_ref[...]) * (g_ref[...] * _BN_INV) + be_ref[...]
        h = jnp.maximum(h, 0.0)
        o = jnp.dot(h, wb_ref[...], preferred_element_type=jnp.float32)
        o = jnp.maximum(o + bb_ref[...], 0.0)
        out_refs[0][...] = o
        if not last:
            out_refs[1][...] = jnp.clip(o + EPS, 1e-7, 1e1)

    Fs = F // nsplit
    out_shape = [jax.ShapeDtypeStruct((N, Fo), jnp.float32)]
    out_specs = [pl.BlockSpec((_BLK, Fo), lambda i: (i, 0))]
    if not last:
        out_shape.append(jax.ShapeDtypeStruct((N, Fo), jnp.float32))
        out_specs.append(pl.BlockSpec((_BLK, Fo), lambda i: (i, 0)))

    return pl.pallas_call(
        body,
        grid=(_GRID,),
        in_specs=(
            [pl.BlockSpec((2, _BLK, Fs), lambda i: (0, i, 0))] * nsplit +
            [
                pl.BlockSpec((_BLK, NW), lambda i: (i, 0)),
                pl.BlockSpec((_BLK, F), lambda i: (i, 0)),
                pl.BlockSpec((F, Fh), lambda i: (0, 0)),
                pl.BlockSpec((1, Fh), lambda i: (0, 0)),
                pl.BlockSpec((1, Fh), lambda i: (0, 0)),
                pl.BlockSpec((1, Fh), lambda i: (0, 0)),
                pl.BlockSpec((Fh, Fo), lambda i: (0, 0)),
                pl.BlockSpec((1, Fo), lambda i: (0, 0)),
            ]
        ),
        out_specs=out_specs,
        out_shape=out_shape,
    )


_layer1_tc = _make_layer_tc(128, 256, 32, last=False)
_layer2_tc = _make_layer_tc(32, 64, 16, last=False)
_layer3_tc = _make_layer_tc(16, 32, 8, last=True)


def kernel(x, edge_index, w1a, b1a, g1, be1, w1b, b1b,
           w2a, b2a, g2, be2, w2b, b2b, w3a, b3a, g3, be3, w3b, b3b):
    if EPAD:
        pad = jnp.stack([jnp.zeros((EPAD,), jnp.int32),
                         jnp.full((EPAD,), PAD_DST, jnp.int32)])
        eidx = jnp.concatenate([edge_index, pad], axis=1)
    else:
        eidx = edge_index
    eidx = eidx.reshape(2, NW, NCHUNK, CHUNK)

    r2 = lambda v: v.reshape(1, -1)

    m1lo, m1hi = _msg_tc(x)
    s1p, degh = _make_sc_scatter(64, 2, True)(m1lo, m1hi, eidx)
    degp = jnp.swapaxes(degh, 0, 1)
    h1, m2 = _layer1_tc(s1p, degp, x, w1a, r2(b1a), r2(g1), r2(be1),
                        w1b, r2(b1b))
    s2p = _make_sc_scatter(32, 1, False)(m2, eidx)[0]
    h2, m3 = _layer2_tc(s2p, degp, h1, w2a, r2(b2a), r2(g2), r2(be2),
                        w2b, r2(b2b))
    s3p = _make_sc_scatter(16, 1, False)(m3, eidx)[0]
    h3 = _layer3_tc(s3p, degp, h2, w3a, r2(b3a), r2(g3), r2(be3),
                    w3b, r2(b3b))[0]
    return h3


# speedup vs baseline: 1.2117x; 1.0852x over previous
import functools

import jax
import jax.numpy as jnp
import numpy as np
from jax import lax
from jax.experimental import pallas as pl
from jax.experimental.pallas import tpu as pltpu
from jax.experimental.pallas import tpu_sc as plsc

N = 10000
E = 320000
EPS = 1e-7

NC = 2
NS = 16
NW = NC * NS
CHUNK = 80
NCHUNK = 125
EPAD = NW * NCHUNK * CHUNK - E
NP = 10240
PAD_DST = 10200
ROWS_PER_SUB = NP // NS
ZROWS = 128

assert NW * CHUNK * NCHUNK == E + EPAD
assert EPAD == 0 or N <= PAD_DST < NP
assert ZROWS * 5 == ROWS_PER_SUB
assert (NCHUNK - 5) % 4 == 0 and NCHUNK >= 9


@functools.lru_cache(maxsize=None)
def _make_sc_scatter(F, nsplit, with_deg):
    mesh = plsc.VectorSubcoreMesh(core_axis_name="c", subcore_axis_name="s",
                                  num_cores=NC, num_subcores=NS)
    out_type = [jax.ShapeDtypeStruct((NC, NP, F * nsplit), jnp.float32)]
    if with_deg:
        out_type.append(jax.ShapeDtypeStruct((NW, NP), jnp.float32))

    scratch_seq = [
        pltpu.VMEM((NCHUNK, CHUNK), jnp.int32),
        pltpu.VMEM((NCHUNK, CHUNK), jnp.int32),
        pltpu.VMEM((CHUNK, F), jnp.float32),
        pltpu.VMEM((CHUNK, F), jnp.float32),
        pltpu.VMEM((CHUNK, F), jnp.float32),
        pltpu.VMEM((CHUNK, F), jnp.float32),
        pltpu.VMEM((CHUNK, F), jnp.float32),
        pltpu.VMEM((CHUNK, F), jnp.float32),
        pltpu.VMEM((ZROWS, F), jnp.float32),
        pltpu.VMEM_SHARED((NP, F), jnp.float32),
        pltpu.SemaphoreType.DMA,
        pltpu.SemaphoreType.DMA,
        pltpu.SemaphoreType.DMA,
        pltpu.SemaphoreType.DMA,
        pltpu.SemaphoreType.DMA,
        pltpu.SemaphoreType.DMA,
        pltpu.SemaphoreType.DMA,
        pltpu.SemaphoreType.DMA,
        pltpu.SemaphoreType.DMA,
        pltpu.SemaphoreType.DMA,
        pltpu.SemaphoreType.DMA,
        pltpu.SemaphoreType.DMA,
    ]
    if with_deg:
        scratch_seq.append(pltpu.VMEM((NP,), jnp.float32))

    def tec_body(*refs):
        m_hbm = refs[:nsplit]
        eidx_hbm = refs[nsplit]
        s_out = refs[nsplit + 1]
        rest = refs[nsplit + 2:]
        if with_deg:
            deg_out = rest[0]
            rest = rest[1:]
        (src_v, dst_v, b0, b1, b2, b3, b4, b5, zbuf, s_sh,
         g0, g1, g2, g3, g4, g5, s0, s1, s2, s3, s4, s5) = rest[:22]
        bufs = [b0, b1, b2, b3, b4, b5]
        gsems = [g0, g1, g2, g3, g4, g5]
        ssems = [s0, s1, s2, s3, s4, s5]
        if with_deg:
            hist = rest[22]

        cid = lax.axis_index("c")
        sid = lax.axis_index("s")
        wid = cid * NS + sid
        base = sid * ROWS_PER_SUB

        zero16 = jnp.zeros((16,), jnp.float32)
        one16 = jnp.ones((16,), jnp.float32)

        def zb_row(r, _):
            for cc in range(F // 16):
                zbuf[r, pl.ds(cc * 16, 16)] = zero16
            return 0
        lax.fori_loop(0, ZROWS, zb_row, 0)
        if with_deg:
            def h_row(r, _):
                hist[pl.ds(r * 16, 16)] = zero16
                return 0
            lax.fori_loop(0, NP // 16, h_row, 0)

        pltpu.sync_copy(eidx_hbm.at[0, wid], src_v)
        pltpu.sync_copy(eidx_hbm.at[1, wid], dst_v)

        for p in range(nsplit):
            first = p == 0
            for t in range(5):
                pltpu.sync_copy(zbuf, s_sh.at[pl.ds(base + t * ZROWS, ZROWS)])
            plsc.subcore_barrier()

            m = m_hbm[p]

            def g_start(c, b):
                pltpu.async_copy(m.at[src_v.at[c]], bufs[b], gsems[b])

            def g_wait(c, b):
                pltpu.make_async_copy(m.at[src_v.at[c]], bufs[b],
                                      gsems[b]).wait()

            def s_start(c, b):
                pltpu.async_copy(bufs[b], s_sh.at[dst_v.at[c]], ssems[b],
                                 add=True)

            def s_wait(c, b):
                pltpu.make_async_copy(bufs[b], s_sh.at[dst_v.at[c]],
                                      ssems[b]).wait()

            def consume(j, b):
                g_wait(j, b)
                s_start(j, b)
                if with_deg and first:
                    for k in range(CHUNK // 16):
                        iv = dst_v[j, pl.ds(k * 16, 16)]
                        plsc.addupdate_scatter(hist, [iv], one16)

            g_start(0, 0)
            g_start(1, 1)
            g_start(2, 2)
            g_start(3, 3)
            consume(0, 0)
            g_start(4, 4)
            consume(1, 1)
            g_start(5, 5)
            consume(2, 2)

            NMAIN = ((NCHUNK - 4) - 3) // 6 * 6

            def hexa(k, _):
                j0 = 6 * k + 3
                for off in range(6):
                    j = j0 + off
                    b = (3 + off) % 6
                    bn = (b + 3) % 6
                    s_wait(j - 3, bn)
                    g_start(j + 3, bn)
                    consume(j, b)
                return 0
            lax.fori_loop(0, NMAIN // 6, hexa, 0)

            for j in range(3 + NMAIN, NCHUNK):
                b = j % 6
                bn = (b + 3) % 6
                s_wait(j - 3, bn)
                if j + 3 < NCHUNK:
                    g_start(j + 3, bn)
                consume(j, b)
            for j in (NCHUNK - 3, NCHUNK - 2, NCHUNK - 1):
                s_wait(j, j % 6)
            plsc.subcore_barrier()

            if nsplit == 1:
                dst_ref = s_out.at[cid, pl.ds(base, ROWS_PER_SUB)]
            else:
                dst_ref = s_out.at[cid, pl.ds(base, ROWS_PER_SUB),
                                   pl.ds(p * F, F)]
            pltpu.sync_copy(s_sh.at[pl.ds(base, ROWS_PER_SUB)], dst_ref)
        if with_deg:
            pltpu.sync_copy(hist, deg_out.at[wid])

    return pl.kernel(
        tec_body,
        out_type=tuple(out_type),
        mesh=mesh,
        scratch_types=tuple(scratch_seq),
        compiler_params=pltpu.CompilerParams(
            use_tc_tiling_on_sc=False,
            needs_layout_passes=not with_deg),
    )


_BLK = 2000
_GRID = N // _BLK
_BN_INV = 1.0 / np.sqrt(1.0 + 1e-5)


def _msg_body(x_ref, lo_ref, hi_ref):
    m = jnp.clip(jnp.maximum(x_ref[...], 0.0) + EPS, 1e-7, 1e1)
    lo_ref[...] = m[:, :64]
    hi_ref[...] = m[:, 64:]


_msg_tc = pl.pallas_call(
    _msg_body,
    grid=(_GRID,),
    in_specs=[pl.BlockSpec((_BLK, 128), lambda i: (i, 0))],
    out_specs=[pl.BlockSpec((_BLK, 64), lambda i: (i, 0)),
               pl.BlockSpec((_BLK, 64), lambda i: (i, 0))],
    out_shape=[jax.ShapeDtypeStruct((N, 64), jnp.float32),
               jax.ShapeDtypeStruct((N, 64), jnp.float32)],
)


def _make_layer_tc(F, Fh, Fo, last, nsplit=1):
    def body(*refs):
        sp = refs[:nsplit]
        (degp_ref, x_ref, wa_ref, ba_ref, g_ref, be_ref, wb_ref,
         bb_ref) = refs[nsplit:nsplit + 8]
        out_refs = refs[nsplit + 8:]
        s = jnp.concatenate([r[0] + r[1] for r in sp], axis=-1)
        deg = jnp.sum(degp_ref[...], axis=1, keepdims=True)
        deg = jnp.maximum(deg, 1.0)
        out = jnp.clip(s / deg, 1e-7, 1e1) + x_ref[...]
        h = jnp.dot(out, wa_ref[...], preferred_element_type=jnp.float32)
        h = (h + ba_ref[...]) * (g_ref[...] * _BN_INV) + be_ref[...]
        h = jnp.maximum(h, 0.0)
        o = jnp.dot(h, wb_ref[...], preferred_element_type=jnp.float32)
        o = jnp.maximum(o + bb_ref[...], 0.0)
        out_refs[0][...] = o
        if not last:
            out_refs[1][...] = jnp.clip(o + EPS, 1e-7, 1e1)

    Fs = F // nsplit
    out_shape = [jax.ShapeDtypeStruct((N, Fo), jnp.float32)]
    out_specs = [pl.BlockSpec((_BLK, Fo), lambda i: (i, 0))]
    if not last:
        out_shape.append(jax.ShapeDtypeStruct((N, Fo), jnp.float32))
        out_specs.append(pl.BlockSpec((_BLK, Fo), lambda i: (i, 0)))

    return pl.pallas_call(
        body,
        grid=(_GRID,),
        in_specs=(
            [pl.BlockSpec((2, _BLK, Fs), lambda i: (0, i, 0))] * nsplit +
            [
                pl.BlockSpec((_BLK, NW), lambda i: (i, 0)),
                pl.BlockSpec((_BLK, F), lambda i: (i, 0)),
                pl.BlockSpec((F, Fh), lambda i: (0, 0)),
                pl.BlockSpec((1, Fh), lambda i: (0, 0)),
                pl.BlockSpec((1, Fh), lambda i: (0, 0)),
                pl.BlockSpec((1, Fh), lambda i: (0, 0)),
                pl.BlockSpec((Fh, Fo), lambda i: (0, 0)),
                pl.BlockSpec((1, Fo), lambda i: (0, 0)),
            ]
        ),
        out_specs=out_specs,
        out_shape=out_shape,
    )


_layer1_tc = _make_layer_tc(128, 256, 32, last=False)
_layer2_tc = _make_layer_tc(32, 64, 16, last=False)
_layer3_tc = _make_layer_tc(16, 32, 8, last=True)


def kernel(x, edge_index, w1a, b1a, g1, be1, w1b, b1b,
           w2a, b2a, g2, be2, w2b, b2b, w3a, b3a, g3, be3, w3b, b3b):
    if EPAD:
        pad = jnp.stack([jnp.zeros((EPAD,), jnp.int32),
                         jnp.full((EPAD,), PAD_DST, jnp.int32)])
        eidx = jnp.concatenate([edge_index, pad], axis=1)
    else:
        eidx = edge_index
    eidx = eidx.reshape(2, NW, NCHUNK, CHUNK)

    r2 = lambda v: v.reshape(1, -1)

    m1lo, m1hi = _msg_tc(x)
    s1p, degh = _make_sc_scatter(64, 2, True)(m1lo, m1hi, eidx)
    degp = jnp.swapaxes(degh, 0, 1)
    h1, m2 = _layer1_tc(s1p, degp, x, w1a, r2(b1a), r2(g1), r2(be1),
                        w1b, r2(b1b))
    s2p = _make_sc_scatter(32, 1, False)(m2, eidx)[0]
    h2, m3 = _layer2_tc(s2p, degp, h1, w2a, r2(b2a), r2(g2), r2(be2),
                        w2b, r2(b2b))
    s3p = _make_sc_scatter(16, 1, False)(m3, eidx)[0]
    h3 = _layer3_tc(s3p, degp, h2, w3a, r2(b3a), r2(g3), r2(be3),
                    w3b, r2(b3b))[0]
    return h3


# speedup vs baseline: 1.2121x; 1.0003x over previous
import functools

import jax
import jax.numpy as jnp
import numpy as np
from jax import lax
from jax.experimental import pallas as pl
from jax.experimental.pallas import tpu as pltpu
from jax.experimental.pallas import tpu_sc as plsc

N = 10000
E = 320000
EPS = 1e-7

NC = 2
NS = 16
NW = NC * NS
CHUNK = 80
NCHUNK = 125
EPAD = NW * NCHUNK * CHUNK - E
NP = 10240
PAD_DST = 10200
ROWS_PER_SUB = NP // NS
ZROWS = 128

LA = 4
NB = 2 * LA

assert NW * CHUNK * NCHUNK == E + EPAD
assert NCHUNK >= 2 * NB
assert EPAD == 0 or N <= PAD_DST < NP
assert ZROWS * 5 == ROWS_PER_SUB
assert (NCHUNK - 5) % 4 == 0 and NCHUNK >= 9


@functools.lru_cache(maxsize=None)
def _make_sc_scatter(F, nsplit, with_deg):
    mesh = plsc.VectorSubcoreMesh(core_axis_name="c", subcore_axis_name="s",
                                  num_cores=NC, num_subcores=NS)
    out_type = [jax.ShapeDtypeStruct((NC, NP, F * nsplit), jnp.float32)]
    if with_deg:
        out_type.append(jax.ShapeDtypeStruct((NW, NP), jnp.float32))

    scratch_seq = [
        pltpu.VMEM((NCHUNK, CHUNK), jnp.int32),
        pltpu.VMEM((NCHUNK, CHUNK), jnp.int32),
    ] + [pltpu.VMEM((CHUNK, F), jnp.float32)] * NB + [
        pltpu.VMEM((ZROWS, F), jnp.float32),
        pltpu.VMEM_SHARED((NP, F), jnp.float32),
    ] + [pltpu.SemaphoreType.DMA] * (2 * NB)
    if with_deg:
        scratch_seq.append(pltpu.VMEM((NP,), jnp.float32))

    def tec_body(*refs):
        m_hbm = refs[:nsplit]
        eidx_hbm = refs[nsplit]
        s_out = refs[nsplit + 1]
        rest = refs[nsplit + 2:]
        if with_deg:
            deg_out = rest[0]
            rest = rest[1:]
        src_v, dst_v = rest[0], rest[1]
        bufs = list(rest[2:2 + NB])
        zbuf, s_sh = rest[2 + NB], rest[3 + NB]
        gsems = list(rest[4 + NB:4 + 2 * NB])
        ssems = list(rest[4 + 2 * NB:4 + 3 * NB])
        if with_deg:
            hist = rest[4 + 3 * NB]

        cid = lax.axis_index("c")
        sid = lax.axis_index("s")
        wid = cid * NS + sid
        base = sid * ROWS_PER_SUB

        zero16 = jnp.zeros((16,), jnp.float32)
        one16 = jnp.ones((16,), jnp.float32)

        def zb_row(r, _):
            for cc in range(F // 16):
                zbuf[r, pl.ds(cc * 16, 16)] = zero16
            return 0
        lax.fori_loop(0, ZROWS, zb_row, 0)
        if with_deg:
            def h_row(r, _):
                hist[pl.ds(r * 16, 16)] = zero16
                return 0
            lax.fori_loop(0, NP // 16, h_row, 0)

        pltpu.sync_copy(eidx_hbm.at[0, wid], src_v)
        pltpu.sync_copy(eidx_hbm.at[1, wid], dst_v)

        for p in range(nsplit):
            first = p == 0
            for t in range(5):
                pltpu.sync_copy(zbuf, s_sh.at[pl.ds(base + t * ZROWS, ZROWS)])
            plsc.subcore_barrier()

            m = m_hbm[p]

            def g_start(c, b):
                pltpu.async_copy(m.at[src_v.at[c]], bufs[b], gsems[b])

            def g_wait(c, b):
                pltpu.make_async_copy(m.at[src_v.at[c]], bufs[b],
                                      gsems[b]).wait()

            def s_start(c, b):
                pltpu.async_copy(bufs[b], s_sh.at[dst_v.at[c]], ssems[b],
                                 add=True)

            def s_wait(c, b):
                pltpu.make_async_copy(bufs[b], s_sh.at[dst_v.at[c]],
                                      ssems[b]).wait()

            def consume(j, b):
                g_wait(j, b)
                s_start(j, b)
                if with_deg and first:
                    for k in range(CHUNK // 16):
                        iv = dst_v[j, pl.ds(k * 16, 16)]
                        plsc.addupdate_scatter(hist, [iv], one16)

            for c in range(LA):
                g_start(c, c)
            for i in range(LA):
                g_start(i + LA, i + LA)
                consume(i, i)

            NMAIN = (NCHUNK - LA - LA) // NB * NB

            def grp(k, _):
                j0 = NB * k + LA
                for off in range(NB):
                    j = j0 + off
                    b = (LA + off) % NB
                    bn = (b + LA) % NB
                    s_wait(j - LA, bn)
                    g_start(j + LA, bn)
                    consume(j, b)
                return 0
            lax.fori_loop(0, NMAIN // NB, grp, 0)

            for j in range(LA + NMAIN, NCHUNK):
                b = j % NB
                bn = (b + LA) % NB
                s_wait(j - LA, bn)
                if j + LA < NCHUNK:
                    g_start(j + LA, bn)
                consume(j, b)
            for j in range(NCHUNK - LA, NCHUNK):
                s_wait(j, j % NB)
            plsc.subcore_barrier()

            if nsplit == 1:
                dst_ref = s_out.at[cid, pl.ds(base, ROWS_PER_SUB)]
            else:
                dst_ref = s_out.at[cid, pl.ds(base, ROWS_PER_SUB),
                                   pl.ds(p * F, F)]
            pltpu.sync_copy(s_sh.at[pl.ds(base, ROWS_PER_SUB)], dst_ref)
        if with_deg:
            pltpu.sync_copy(hist, deg_out.at[wid])

    return pl.kernel(
        tec_body,
        out_type=tuple(out_type),
        mesh=mesh,
        scratch_types=tuple(scratch_seq),
        compiler_params=pltpu.CompilerParams(
            use_tc_tiling_on_sc=False,
            needs_layout_passes=not with_deg),
    )


_BLK = 2000
_GRID = N // _BLK
_BN_INV = 1.0 / np.sqrt(1.0 + 1e-5)


def _msg_body(x_ref, lo_ref, hi_ref):
    m = jnp.clip(jnp.maximum(x_ref[...], 0.0) + EPS, 1e-7, 1e1)
    lo_ref[...] = m[:, :64]
    hi_ref[...] = m[:, 64:]


_msg_tc = pl.pallas_call(
    _msg_body,
    grid=(_GRID,),
    in_specs=[pl.BlockSpec((_BLK, 128), lambda i: (i, 0))],
    out_specs=[pl.BlockSpec((_BLK, 64), lambda i: (i, 0)),
               pl.BlockSpec((_BLK, 64), lambda i: (i, 0))],
    out_shape=[jax.ShapeDtypeStruct((N, 64), jnp.float32),
               jax.ShapeDtypeStruct((N, 64), jnp.float32)],
)


def _make_layer_tc(F, Fh, Fo, last, nsplit=1):
    def body(*refs):
        sp = refs[:nsplit]
        (degp_ref, x_ref, wa_ref, ba_ref, g_ref, be_ref, wb_ref,
         bb_ref) = refs[nsplit:nsplit + 8]
        out_refs = refs[nsplit + 8:]
        s = jnp.concatenate([r[0] + r[1] for r in sp], axis=-1)
        deg = jnp.sum(degp_ref[...], axis=1, keepdims=True)
        deg = jnp.maximum(deg, 1.0)
        out = jnp.clip(s / deg, 1e-7, 1e1) + x_ref[...]
        h = jnp.dot(out, wa_ref[...], preferred_element_type=jnp.float32)
        h = (h + ba_ref[...]) * (g_ref[...] * _BN_INV) + be_ref[...]
        h = jnp.maximum(h, 0.0)
        o = jnp.dot(h, wb_ref[...], preferred_element_type=jnp.float32)
        o = jnp.maximum(o + bb_ref[...], 0.0)
        out_refs[0][...] = o
        if not last:
            out_refs[1][...] = jnp.clip(o + EPS, 1e-7, 1e1)

    Fs = F // nsplit
    out_shape = [jax.ShapeDtypeStruct((N, Fo), jnp.float32)]
    out_specs = [pl.BlockSpec((_BLK, Fo), lambda i: (i, 0))]
    if not last:
        out_shape.append(jax.ShapeDtypeStruct((N, Fo), jnp.float32))
        out_specs.append(pl.BlockSpec((_BLK, Fo), lambda i: (i, 0)))

    return pl.pallas_call(
        body,
        grid=(_GRID,),
        in_specs=(
            [pl.BlockSpec((2, _BLK, Fs), lambda i: (0, i, 0))] * nsplit +
            [
                pl.BlockSpec((_BLK, NW), lambda i: (i, 0)),
                pl.BlockSpec((_BLK, F), lambda i: (i, 0)),
                pl.BlockSpec((F, Fh), lambda i: (0, 0)),
                pl.BlockSpec((1, Fh), lambda i: (0, 0)),
                pl.BlockSpec((1, Fh), lambda i: (0, 0)),
                pl.BlockSpec((1, Fh), lambda i: (0, 0)),
                pl.BlockSpec((Fh, Fo), lambda i: (0, 0)),
                pl.BlockSpec((1, Fo), lambda i: (0, 0)),
            ]
        ),
        out_specs=out_specs,
        out_shape=out_shape,
    )


_layer1_tc = _make_layer_tc(128, 256, 32, last=False)
_layer2_tc = _make_layer_tc(32, 64, 16, last=False)
_layer3_tc = _make_layer_tc(16, 32, 8, last=True)


def kernel(x, edge_index, w1a, b1a, g1, be1, w1b, b1b,
           w2a, b2a, g2, be2, w2b, b2b, w3a, b3a, g3, be3, w3b, b3b):
    if EPAD:
        pad = jnp.stack([jnp.zeros((EPAD,), jnp.int32),
                         jnp.full((EPAD,), PAD_DST, jnp.int32)])
        eidx = jnp.concatenate([edge_index, pad], axis=1)
    else:
        eidx = edge_index
    eidx = eidx.reshape(2, NW, NCHUNK, CHUNK)

    r2 = lambda v: v.reshape(1, -1)

    m1lo, m1hi = _msg_tc(x)
    s1p, degh = _make_sc_scatter(64, 2, True)(m1lo, m1hi, eidx)
    degp = jnp.swapaxes(degh, 0, 1)
    h1, m2 = _layer1_tc(s1p, degp, x, w1a, r2(b1a), r2(g1), r2(be1),
                        w1b, r2(b1b))
    s2p = _make_sc_scatter(32, 1, False)(m2, eidx)[0]
    h2, m3 = _layer2_tc(s2p, degp, h1, w2a, r2(b2a), r2(g2), r2(be2),
                        w2b, r2(b2b))
    s3p = _make_sc_scatter(16, 1, False)(m3, eidx)[0]
    h3 = _layer3_tc(s3p, degp, h2, w3a, r2(b3a), r2(g3), r2(be3),
                    w3b, r2(b3b))[0]
    return h3
